# baseline (speedup 1.0000x reference)
import jax
import jax.numpy as jnp
from jax.experimental import pallas as pl
from jax.experimental.pallas import tpu as pltpu

N_PROXIES = 8192
N_CAMS = 8
PPC = N_PROXIES // N_CAMS
TEMP = 0.07
B = 1024
D = 256
TILE = 128
P = 2048
NT = P // TILE


def _tile_kernel(scal_ref, feat_ref, mem_ref, cam_ref, tgt_ref, out_ref):
    feat = feat_ref[...]
    camv = cam_ref[...]
    camsub = jax.lax.broadcasted_iota(jnp.int32, (N_CAMS, B), 0)
    ohi = (camsub == camv).astype(jnp.float32)
    cnt = jnp.sum(ohi, axis=1, keepdims=True)
    padded = jnp.floor((cnt + (TILE - 1)) * (1.0 / TILE)) * TILE
    r8 = jax.lax.broadcasted_iota(jnp.int32, (N_CAMS, N_CAMS), 0)
    c8 = jax.lax.broadcasted_iota(jnp.int32, (N_CAMS, N_CAMS), 1)
    strict_lt = (c8 < r8).astype(jnp.float32)
    starts = jax.lax.dot_general(
        strict_lt, padded, (((1,), (0,)), ((), ())),
        preferred_element_type=jnp.float32)
    ri = jax.lax.broadcasted_iota(jnp.int32, (B, B), 0)
    ci = jax.lax.broadcasted_iota(jnp.int32, (B, B), 1)
    lt_inc = (ri <= ci).astype(jnp.float32)
    incl = jax.lax.dot_general(
        ohi, lt_inc, (((1,), (0,)), ((), ())),
        preferred_element_type=jnp.float32)
    rank = jnp.sum(ohi * (incl - 1.0), axis=0, keepdims=True)
    pos = jnp.sum(ohi * starts, axis=0, keepdims=True) + rank
    winv = jnp.sum(jnp.where(ohi > 0, 1.0 / cnt, 0.0), axis=0,
                   keepdims=True)
    tgtf = tgt_ref[...].astype(jnp.float32)

    def body(t, acc):
        c = scal_ref[t]
        pj = (jax.lax.broadcasted_iota(jnp.int32, (TILE, 1), 0)
              + t * TILE).astype(jnp.float32)
        gb = pos == pj
        g = gb.astype(jnp.bfloat16)
        x = jax.lax.dot_general(
            g, feat, (((1,), (0,)), ((), ())),
            preferred_element_type=jnp.float32).astype(jnp.bfloat16)
        tgt_t = jnp.sum(jnp.where(gb, tgtf, 0.0), axis=1, keepdims=True)
        w_t = jnp.sum(jnp.where(gb, winv, 0.0), axis=1, keepdims=True)
        w = mem_ref[pl.ds(c * PPC, PPC), :]
        sim = jax.lax.dot_general(
            x, w, (((1,), (1,)), ((), ())), preferred_element_type=jnp.float32
        ) * (1.0 / TEMP)
        lse = jnp.log(jnp.sum(jnp.exp(sim), axis=1, keepdims=True))
        cols = jax.lax.broadcasted_iota(jnp.int32, (TILE, PPC), 1)
        tlogit = jnp.sum(jnp.where(cols == tgt_t.astype(jnp.int32), sim, 0.0),
                         axis=1, keepdims=True)
        return acc + (lse - tlogit) * w_t

    n_real = scal_ref[NT]
    acc = jax.lax.fori_loop(0, n_real, body, jnp.zeros((TILE, 1), jnp.float32))
    lane = jax.lax.broadcasted_iota(jnp.int32, (1, 128), 1)
    out_ref[...] = jnp.where(lane == 0, jnp.sum(acc), 0.0)


def kernel(batch_feat, abs_proxy_label, camid, pseudo_cluster_label, memory,
           epoch, k, inter_loss_epoch):
    camid = camid.astype(jnp.int32)
    local_tgt = (abs_proxy_label % PPC).astype(jnp.int32)

    cams = jnp.arange(N_CAMS, dtype=jnp.int32)
    cnt = jnp.sum((camid[None, :] == cams[:, None]).astype(jnp.int32), axis=1)
    padded = ((cnt + TILE - 1) // TILE) * TILE
    ends = jnp.sum(jnp.where(cams[None, :] <= cams[:, None], padded[None, :], 0),
                   axis=1)
    tile_start = jnp.arange(NT, dtype=jnp.int32) * TILE
    tile_cam = jnp.minimum(
        jnp.sum((tile_start[:, None] >= ends[None, :]).astype(jnp.int32),
                axis=1), N_CAMS - 1)
    n_real = ends[N_CAMS - 1] // TILE
    scalars = jnp.concatenate([tile_cam, n_real[None]]).astype(jnp.int32)

    out = pl.pallas_call(
        _tile_kernel,
        grid_spec=pltpu.PrefetchScalarGridSpec(
            num_scalar_prefetch=1,
            grid=(1,),
            in_specs=[
                pl.BlockSpec((B, D), lambda i, tc: (0, 0)),
                pl.BlockSpec((N_PROXIES, D), lambda i, tc: (0, 0)),
                pl.BlockSpec((1, B), lambda i, tc: (0, 0)),
                pl.BlockSpec((1, B), lambda i, tc: (0, 0)),
            ],
            out_specs=pl.BlockSpec((1, 128), lambda i, tc: (0, 0)),
        ),
        out_shape=jax.ShapeDtypeStruct((1, 128), jnp.float32),
    )(scalars, batch_feat.astype(jnp.bfloat16), memory.astype(jnp.bfloat16),
      camid.reshape(1, B), local_tgt.reshape(1, B))
    return out[0, 0]


# speedup vs baseline: 1.3243x; 1.3243x over previous
import jax
import jax.numpy as jnp
from jax.experimental import pallas as pl
from jax.experimental.pallas import tpu as pltpu

N_PROXIES = 8192
N_CAMS = 8
PPC = N_PROXIES // N_CAMS
TEMP = 0.07
B = 1024
D = 256
TILE = 128
P = 2048
NT = P // TILE


def _tile_kernel(scal_ref, feat_ref, mem_ref, cam_ref, tgt_ref, out_ref):
    feat = feat_ref[...]
    camv = cam_ref[...]
    camsub = jax.lax.broadcasted_iota(jnp.int32, (N_CAMS, B), 0)
    ohi = (camsub == camv).astype(jnp.float32)
    cnt = jnp.sum(ohi, axis=1, keepdims=True)
    padded = jnp.floor((cnt + (TILE - 1)) * (1.0 / TILE)) * TILE
    r8 = jax.lax.broadcasted_iota(jnp.int32, (N_CAMS, N_CAMS), 0)
    c8 = jax.lax.broadcasted_iota(jnp.int32, (N_CAMS, N_CAMS), 1)
    strict_lt = (c8 < r8).astype(jnp.float32)
    starts = jax.lax.dot_general(
        strict_lt, padded, (((1,), (0,)), ((), ())),
        preferred_element_type=jnp.float32)
    ri = jax.lax.broadcasted_iota(jnp.int32, (B, B), 0)
    ci = jax.lax.broadcasted_iota(jnp.int32, (B, B), 1)
    lt_inc = (ri <= ci).astype(jnp.float32)
    incl = jax.lax.dot_general(
        ohi, lt_inc, (((1,), (0,)), ((), ())),
        preferred_element_type=jnp.float32)
    rank = jnp.sum(ohi * (incl - 1.0), axis=0, keepdims=True)
    pos = jnp.sum(ohi * starts, axis=0, keepdims=True) + rank
    winv = jnp.sum(jnp.where(ohi > 0, 1.0 / cnt, 0.0), axis=0,
                   keepdims=True)
    tgtf = tgt_ref[...].astype(jnp.float32)

    def body(t, acc):
        c = scal_ref[t]
        pj = (jax.lax.broadcasted_iota(jnp.int32, (TILE, 1), 0)
              + t * TILE).astype(jnp.float32)
        gb = pos == pj
        g = gb.astype(jnp.float32)
        x = jax.lax.dot_general(
            g, feat, (((1,), (0,)), ((), ())),
            preferred_element_type=jnp.float32)
        tgt_t = jnp.sum(jnp.where(gb, tgtf, 0.0), axis=1, keepdims=True)
        w_t = jnp.sum(jnp.where(gb, winv, 0.0), axis=1, keepdims=True)
        w = mem_ref[pl.ds(c * PPC, PPC), :]
        sim = jax.lax.dot_general(
            x, w, (((1,), (1,)), ((), ())), preferred_element_type=jnp.float32
        ) * (1.0 / TEMP)
        lse = jnp.log(jnp.sum(jnp.exp(sim), axis=1, keepdims=True))
        cols = jax.lax.broadcasted_iota(jnp.int32, (TILE, PPC), 1)
        tlogit = jnp.sum(jnp.where(cols == tgt_t.astype(jnp.int32), sim, 0.0),
                         axis=1, keepdims=True)
        return acc + (lse - tlogit) * w_t

    n_real = scal_ref[NT]
    acc = jax.lax.fori_loop(0, n_real, body, jnp.zeros((TILE, 1), jnp.float32))
    lane = jax.lax.broadcasted_iota(jnp.int32, (1, 128), 1)
    out_ref[...] = jnp.where(lane == 0, jnp.sum(acc), 0.0)


def kernel(batch_feat, abs_proxy_label, camid, pseudo_cluster_label, memory,
           epoch, k, inter_loss_epoch):
    camid = camid.astype(jnp.int32)
    local_tgt = (abs_proxy_label % PPC).astype(jnp.int32)

    cams = jnp.arange(N_CAMS, dtype=jnp.int32)
    cnt = jnp.sum((camid[None, :] == cams[:, None]).astype(jnp.int32), axis=1)
    padded = ((cnt + TILE - 1) // TILE) * TILE
    ends = jnp.sum(jnp.where(cams[None, :] <= cams[:, None], padded[None, :], 0),
                   axis=1)
    tile_start = jnp.arange(NT, dtype=jnp.int32) * TILE
    tile_cam = jnp.minimum(
        jnp.sum((tile_start[:, None] >= ends[None, :]).astype(jnp.int32),
                axis=1), N_CAMS - 1)
    n_real = ends[N_CAMS - 1] // TILE
    scalars = jnp.concatenate([tile_cam, n_real[None]]).astype(jnp.int32)

    out = pl.pallas_call(
        _tile_kernel,
        grid_spec=pltpu.PrefetchScalarGridSpec(
            num_scalar_prefetch=1,
            grid=(1,),
            in_specs=[
                pl.BlockSpec((B, D), lambda i, tc: (0, 0)),
                pl.BlockSpec((N_PROXIES, D), lambda i, tc: (0, 0)),
                pl.BlockSpec((1, B), lambda i, tc: (0, 0)),
                pl.BlockSpec((1, B), lambda i, tc: (0, 0)),
            ],
            out_specs=pl.BlockSpec((1, 128), lambda i, tc: (0, 0)),
        ),
        out_shape=jax.ShapeDtypeStruct((1, 128), jnp.float32),
    )(scalars, batch_feat, memory,
      camid.reshape(1, B), local_tgt.reshape(1, B))
    return out[0, 0]
